# baseline (device time: 208709 ns/iter reference)
import jax
import jax.numpy as jnp
from jax import lax
from jax.experimental import pallas as pl
from jax.experimental.pallas import tpu as pltpu

M = 4096
D = 4096
CH = M // 4
NB = 8
BR = CH // NB
TR = BR
H2 = BR // 2
TILE = 128
EPS = 1e-6

_MESH = pl.DeviceIdType.MESH


def _alloc_bf16():

    def body(o_ref):
        pass

    return pl.pallas_call(
        body,
        out_shape=jax.ShapeDtypeStruct((M, D), jnp.bfloat16),
        out_specs=pl.BlockSpec(memory_space=pl.ANY),
    )()


def _fused(partial, resid, gamma2d, gbuf_in):
    def body(
        p_ref, res_ref, g_ref, gbuf_in_ref,
        out_ref, gbuf_ref,
        recv_y, gamma_v, pstage, rstage, ntile, cvt_in, cvt_out,
        ysend, yrecv, xs, xr, zs, zr, fxs, fxr, fzs, fzr,
        gsem, psem, rsem, nsem, osem, cisem,
    ):
        del gbuf_in_ref
        my_x = lax.axis_index("x")
        my_y = lax.axis_index("y")
        my_z = lax.axis_index("z")
        nbr_y = (my_x, 1 - my_y, my_z)
        nbr_x = (1 - my_x, my_y, my_z)
        nbr_z = (my_x, my_y, 1 - my_z)

        c0 = (my_x * 2 + my_z) * CH
        cx0 = ((1 - my_x) * 2 + my_z) * CH
        cz0 = (my_x * 2 + (1 - my_z)) * CH
        cd0 = ((1 - my_x) * 2 + (1 - my_z)) * CH

        barrier_sem = pltpu.get_barrier_semaphore()
        for nbr in (nbr_x, nbr_y, nbr_z):
            pl.semaphore_signal(
                barrier_sem, inc=1, device_id=nbr, device_id_type=_MESH
            )
        pl.semaphore_wait(barrier_sem, 3)

        def rcopy(src, dst, ssem, rsem_, nbr):
            return pltpu.make_async_remote_copy(
                src_ref=src, dst_ref=dst, send_sem=ssem, recv_sem=rsem_,
                device_id=nbr, device_id_type=_MESH,
            )

        gdma = pltpu.make_async_copy(g_ref, gamma_v, gsem)
        gdma.start()
        gdma.wait()

        y_rdmas, x_rdmas, z_rdmas = [], [], []
        rds = [None] * NB
        pds = [None] * NB
        od_prev = [None, None]

        def start_pr(b):
            slot = b % 2
            pds[b] = pltpu.make_async_copy(
                p_ref.at[0, pl.ds(c0 + b * BR, BR)], pstage.at[slot], psem.at[slot]
            )
            rds[b] = pltpu.make_async_copy(
                res_ref.at[pl.ds(c0 + b * BR, BR)], rstage.at[slot], rsem.at[slot]
            )
            pds[b].start()
            rds[b].start()

        def stage_cast_send(b):
            slot = b % 2
            if b >= 2:
                y_rdmas[b - 2].wait_send()
            pds[b].wait()
            cvt_in[slot] = pstage[slot].astype(jnp.bfloat16)
            r = rcopy(
                cvt_in.at[slot], recv_y.at[b],
                ysend.at[b], yrecv.at[b], nbr_y,
            )
            r.start()
            y_rdmas.append(r)

        def finish_block(b):
            slot = b % 2
            y_rdmas[b].wait_recv()
            rds[b].wait()
            y32 = pstage[slot] + recv_y[b].astype(jnp.float32) + rstage[slot]
            if b + 2 < NB:
                start_pr(b + 2)
            ms = jnp.mean(y32 * y32, axis=-1, keepdims=True)
            norm = y32 * lax.rsqrt(ms + EPS) * gamma_v[...]
            if od_prev[slot] is not None:
                od_prev[slot].wait()
            cvt_out[slot] = norm
            ntile[slot] = norm.astype(jnp.bfloat16)
            od = pltpu.make_async_copy(
                cvt_out.at[slot], out_ref.at[pl.ds(c0 + b * BR, BR)], osem.at[slot]
            )
            nd = pltpu.make_async_copy(
                ntile.at[slot], gbuf_ref.at[pl.ds(c0 + b * BR, BR)], nsem.at[slot]
            )
            od.start()
            nd.start()
            od_prev[slot] = od
            nd.wait()
            rx = rcopy(
                gbuf_ref.at[pl.ds(c0 + b * BR, BR)],
                gbuf_ref.at[pl.ds(c0 + b * BR, BR)],
                xs.at[b], xr.at[b], nbr_x,
            )
            rz = rcopy(
                gbuf_ref.at[pl.ds(c0 + b * BR, BR)],
                gbuf_ref.at[pl.ds(c0 + b * BR, BR)],
                zs.at[b], zr.at[b], nbr_z,
            )
            rx.start()
            rz.start()
            x_rdmas.append(rx)
            z_rdmas.append(rz)

        start_pr(0)
        start_pr(1)
        stage_cast_send(0)
        for b in range(1, NB):
            stage_cast_send(b)
            finish_block(b - 1)
        finish_block(NB - 1)
        for slot in (0, 1):
            if od_prev[slot] is not None:
                od_prev[slot].wait()

        y_rdmas[NB - 2].wait_send()
        y_rdmas[NB - 1].wait_send()

        cvt_queue = []
        cvt_state = {"in": [None, None], "out": [None, None], "slot": 0}

        def _cvt_drain_out(slot):
            if cvt_state["out"][slot] is not None:
                cvt_state["out"][slot].wait()
                cvt_state["out"][slot] = None

        def _cvt_process(slot):
            if cvt_state["in"][slot] is None:
                return
            desc, r0 = cvt_state["in"][slot]
            desc.wait()
            cvt_state["in"][slot] = None
            cvt_out[slot] = cvt_in[slot].astype(jnp.float32)
            od = pltpu.make_async_copy(
                cvt_out.at[slot], out_ref.at[pl.ds(r0, TR)], osem.at[slot]
            )
            od.start()
            cvt_state["out"][slot] = od

        def convert_some(n):
            for _ in range(n):
                if not cvt_queue:
                    return
                r0 = cvt_queue.pop(0)
                slot = cvt_state["slot"]
                cvt_state["slot"] = 1 - slot
                _cvt_process(slot)
                _cvt_drain_out(slot)
                ci = pltpu.make_async_copy(
                    gbuf_ref.at[pl.ds(r0, TR)], cvt_in.at[slot], cisem.at[slot]
                )
                ci.start()
                cvt_state["in"][slot] = (ci, r0)
                _cvt_process(1 - slot)

        def convert_flush():
            convert_some(len(cvt_queue))
            for slot in (0, 1):
                _cvt_process(slot)
            for slot in (0, 1):
                _cvt_drain_out(slot)

        fx_rdmas, fz_rdmas = [], []
        for b in range(NB):
            rcopy(
                gbuf_ref.at[pl.ds(cz0 + b * BR, BR)],
                gbuf_ref.at[pl.ds(cz0 + b * BR, BR)],
                zs.at[b], zr.at[b], nbr_z,
            ).wait_recv()
            fx = rcopy(
                gbuf_ref.at[pl.ds(cz0 + b * BR, H2)],
                gbuf_ref.at[pl.ds(cz0 + b * BR, H2)],
                fxs.at[b], fxr.at[b], nbr_x,
            )
            fx.start()
            fx_rdmas.append(fx)
            rcopy(
                gbuf_ref.at[pl.ds(cx0 + b * BR, BR)],
                gbuf_ref.at[pl.ds(cx0 + b * BR, BR)],
                xs.at[b], xr.at[b], nbr_x,
            ).wait_recv()
            fz = rcopy(
                gbuf_ref.at[pl.ds(cx0 + b * BR + H2, H2)],
                gbuf_ref.at[pl.ds(cx0 + b * BR + H2, H2)],
                fzs.at[b], fzr.at[b], nbr_z,
            )
            fz.start()
            fz_rdmas.append(fz)
            cvt_queue.append(cz0 + b * BR)
            cvt_queue.append(cx0 + b * BR)
            convert_some(2)

        for b in range(NB):
            rcopy(
                gbuf_ref.at[pl.ds(cd0 + b * BR, H2)],
                gbuf_ref.at[pl.ds(cd0 + b * BR, H2)],
                fxs.at[b], fxr.at[b], nbr_x,
            ).wait_recv()
            rcopy(
                gbuf_ref.at[pl.ds(cd0 + b * BR + H2, H2)],
                gbuf_ref.at[pl.ds(cd0 + b * BR + H2, H2)],
                fzs.at[b], fzr.at[b], nbr_z,
            ).wait_recv()
            cvt_queue.append(cd0 + b * BR)
            convert_some(1)
        convert_flush()

        for r in x_rdmas + z_rdmas + fx_rdmas + fz_rdmas:
            r.wait_send()

    out_f32, _gbuf = pl.pallas_call(
        body,
        out_shape=[
            jax.ShapeDtypeStruct((M, D), jnp.float32),
            jax.ShapeDtypeStruct((M, D), jnp.bfloat16),
        ],
        in_specs=[pl.BlockSpec(memory_space=pl.ANY)] * 4,
        out_specs=[pl.BlockSpec(memory_space=pl.ANY)] * 2,
        input_output_aliases={3: 1},
        scratch_shapes=[
            pltpu.VMEM((NB, BR, D), jnp.bfloat16),
            pltpu.VMEM((1, D), jnp.float32),
            pltpu.VMEM((2, TR, D), jnp.float32),
            pltpu.VMEM((2, TR, D), jnp.float32),
            pltpu.VMEM((2, TR, D), jnp.bfloat16),
            pltpu.VMEM((2, TR, D), jnp.bfloat16),
            pltpu.VMEM((2, TR, D), jnp.float32),
        ]
        + [pltpu.SemaphoreType.DMA((NB,))] * 10
        + [
            pltpu.SemaphoreType.DMA,
            pltpu.SemaphoreType.DMA((2,)),
            pltpu.SemaphoreType.DMA((2,)),
            pltpu.SemaphoreType.DMA((2,)),
            pltpu.SemaphoreType.DMA((2,)),
            pltpu.SemaphoreType.DMA((2,)),
        ],
        compiler_params=pltpu.CompilerParams(collective_id=0),
    )(partial, resid, gamma2d, gbuf_in)
    return out_f32


def kernel(partial, resid, gamma):
    return _fused(partial, resid, gamma.reshape(1, D), _alloc_bf16())


# device time: 207576 ns/iter; 1.0055x vs baseline; 1.0055x over previous
import jax
import jax.numpy as jnp
from jax import lax
from jax.experimental import pallas as pl
from jax.experimental.pallas import tpu as pltpu

M = 4096
D = 4096
CH = M // 4
NB = 8
BR = CH // NB
TR = BR
H2 = BR // 2
TILE = 128
EPS = 1e-6

_MESH = pl.DeviceIdType.MESH


def _alloc(dtype):

    def body(o_ref):
        pass

    return pl.pallas_call(
        body,
        out_shape=jax.ShapeDtypeStruct((M, D), dtype),
        out_specs=pl.BlockSpec(memory_space=pl.ANY),
    )()


def _fused(partial, resid, gamma2d, out_in, gbuf_in):
    def body(
        p_ref, res_ref, g_ref, out_in_ref, gbuf_in_ref,
        out_ref, gbuf_ref,
        recv_y, gamma_v, pstage, rstage, ntile, cvt_in, cvt_out,
        ysend, yrecv, xs, xr, zs, zr, fxs, fxr, fzs, fzr,
        gsem, psem, rsem, nsem, osem, cisem,
    ):
        del out_in_ref, gbuf_in_ref
        my_x = lax.axis_index("x")
        my_y = lax.axis_index("y")
        my_z = lax.axis_index("z")
        nbr_y = (my_x, 1 - my_y, my_z)
        nbr_x = (1 - my_x, my_y, my_z)
        nbr_z = (my_x, my_y, 1 - my_z)

        c0 = (my_x * 2 + my_z) * CH
        cx0 = ((1 - my_x) * 2 + my_z) * CH
        cz0 = (my_x * 2 + (1 - my_z)) * CH
        cd0 = ((1 - my_x) * 2 + (1 - my_z)) * CH

        barrier_sem = pltpu.get_barrier_semaphore()
        for nbr in (nbr_x, nbr_y, nbr_z):
            pl.semaphore_signal(
                barrier_sem, inc=1, device_id=nbr, device_id_type=_MESH
            )
        pl.semaphore_wait(barrier_sem, 3)

        def rcopy(src, dst, ssem, rsem_, nbr):
            return pltpu.make_async_remote_copy(
                src_ref=src, dst_ref=dst, send_sem=ssem, recv_sem=rsem_,
                device_id=nbr, device_id_type=_MESH,
            )

        gdma = pltpu.make_async_copy(g_ref, gamma_v, gsem)
        gdma.start()
        gdma.wait()

        y_rdmas, x_rdmas, z_rdmas = [], [], []
        rds = [None] * NB
        pds = [None] * NB
        od_prev = [None, None]

        def start_pr(b):
            slot = b % 2
            pds[b] = pltpu.make_async_copy(
                p_ref.at[0, pl.ds(c0 + b * BR, BR)], pstage.at[slot], psem.at[slot]
            )
            rds[b] = pltpu.make_async_copy(
                res_ref.at[pl.ds(c0 + b * BR, BR)], rstage.at[slot], rsem.at[slot]
            )
            pds[b].start()
            rds[b].start()

        def stage_cast_send(b):
            slot = b % 2
            if b >= 2:
                y_rdmas[b - 2].wait_send()
            pds[b].wait()
            cvt_in[slot] = pstage[slot].astype(jnp.bfloat16)
            r = rcopy(
                cvt_in.at[slot], recv_y.at[b],
                ysend.at[b], yrecv.at[b], nbr_y,
            )
            r.start()
            y_rdmas.append(r)

        def finish_block(b):
            slot = b % 2
            y_rdmas[b].wait_recv()
            rds[b].wait()
            y32 = pstage[slot] + recv_y[b].astype(jnp.float32) + rstage[slot]
            if b + 2 < NB:
                start_pr(b + 2)
            ms = jnp.mean(y32 * y32, axis=-1, keepdims=True)
            norm = y32 * lax.rsqrt(ms + EPS) * gamma_v[...]
            if od_prev[slot] is not None:
                od_prev[slot].wait()
            cvt_out[slot] = norm
            ntile[slot] = norm.astype(jnp.bfloat16)
            od = pltpu.make_async_copy(
                cvt_out.at[slot], out_ref.at[pl.ds(c0 + b * BR, BR)], osem.at[slot]
            )
            nd = pltpu.make_async_copy(
                ntile.at[slot], gbuf_ref.at[pl.ds(c0 + b * BR, BR)], nsem.at[slot]
            )
            od.start()
            nd.start()
            od_prev[slot] = od
            nd.wait()
            rx = rcopy(
                gbuf_ref.at[pl.ds(c0 + b * BR, BR)],
                gbuf_ref.at[pl.ds(c0 + b * BR, BR)],
                xs.at[b], xr.at[b], nbr_x,
            )
            rz = rcopy(
                gbuf_ref.at[pl.ds(c0 + b * BR, BR)],
                gbuf_ref.at[pl.ds(c0 + b * BR, BR)],
                zs.at[b], zr.at[b], nbr_z,
            )
            rx.start()
            rz.start()
            x_rdmas.append(rx)
            z_rdmas.append(rz)

        start_pr(0)
        start_pr(1)
        stage_cast_send(0)
        for b in range(1, NB):
            stage_cast_send(b)
            finish_block(b - 1)
        finish_block(NB - 1)
        for slot in (0, 1):
            if od_prev[slot] is not None:
                od_prev[slot].wait()

        y_rdmas[NB - 2].wait_send()
        y_rdmas[NB - 1].wait_send()

        cvt_queue = []
        cvt_state = {"in": [None, None], "out": [None, None], "slot": 0}

        def _cvt_drain_out(slot):
            if cvt_state["out"][slot] is not None:
                cvt_state["out"][slot].wait()
                cvt_state["out"][slot] = None

        def _cvt_process(slot):
            if cvt_state["in"][slot] is None:
                return
            desc, r0 = cvt_state["in"][slot]
            desc.wait()
            cvt_state["in"][slot] = None
            cvt_out[slot] = cvt_in[slot].astype(jnp.float32)
            od = pltpu.make_async_copy(
                cvt_out.at[slot], out_ref.at[pl.ds(r0, TR)], osem.at[slot]
            )
            od.start()
            cvt_state["out"][slot] = od

        def convert_some(n):
            for _ in range(n):
                if not cvt_queue:
                    return
                r0 = cvt_queue.pop(0)
                slot = cvt_state["slot"]
                cvt_state["slot"] = 1 - slot
                _cvt_process(slot)
                _cvt_drain_out(slot)
                ci = pltpu.make_async_copy(
                    gbuf_ref.at[pl.ds(r0, TR)], cvt_in.at[slot], cisem.at[slot]
                )
                ci.start()
                cvt_state["in"][slot] = (ci, r0)
                _cvt_process(1 - slot)

        def convert_flush():
            convert_some(len(cvt_queue))
            for slot in (0, 1):
                _cvt_process(slot)
            for slot in (0, 1):
                _cvt_drain_out(slot)

        fx_rdmas, fz_rdmas = [], []
        for b in range(NB):
            rcopy(
                gbuf_ref.at[pl.ds(cz0 + b * BR, BR)],
                gbuf_ref.at[pl.ds(cz0 + b * BR, BR)],
                zs.at[b], zr.at[b], nbr_z,
            ).wait_recv()
            fx = rcopy(
                gbuf_ref.at[pl.ds(cz0 + b * BR, H2)],
                gbuf_ref.at[pl.ds(cz0 + b * BR, H2)],
                fxs.at[b], fxr.at[b], nbr_x,
            )
            fx.start()
            fx_rdmas.append(fx)
            rcopy(
                gbuf_ref.at[pl.ds(cx0 + b * BR, BR)],
                gbuf_ref.at[pl.ds(cx0 + b * BR, BR)],
                xs.at[b], xr.at[b], nbr_x,
            ).wait_recv()
            fz = rcopy(
                gbuf_ref.at[pl.ds(cx0 + b * BR + H2, H2)],
                gbuf_ref.at[pl.ds(cx0 + b * BR + H2, H2)],
                fzs.at[b], fzr.at[b], nbr_z,
            )
            fz.start()
            fz_rdmas.append(fz)
            cvt_queue.append(cz0 + b * BR)
            cvt_queue.append(cx0 + b * BR)
            convert_some(2)

        for b in range(NB):
            rcopy(
                gbuf_ref.at[pl.ds(cd0 + b * BR, H2)],
                gbuf_ref.at[pl.ds(cd0 + b * BR, H2)],
                fxs.at[b], fxr.at[b], nbr_x,
            ).wait_recv()
            rcopy(
                gbuf_ref.at[pl.ds(cd0 + b * BR + H2, H2)],
                gbuf_ref.at[pl.ds(cd0 + b * BR + H2, H2)],
                fzs.at[b], fzr.at[b], nbr_z,
            ).wait_recv()
            cvt_queue.append(cd0 + b * BR)
            convert_some(1)
        convert_flush()

        for r in x_rdmas + z_rdmas + fx_rdmas + fz_rdmas:
            r.wait_send()

    out_f32, _gbuf = pl.pallas_call(
        body,
        out_shape=[
            jax.ShapeDtypeStruct((M, D), jnp.float32),
            jax.ShapeDtypeStruct((M, D), jnp.bfloat16),
        ],
        in_specs=[pl.BlockSpec(memory_space=pl.ANY)] * 5,
        out_specs=[pl.BlockSpec(memory_space=pl.ANY)] * 2,
        input_output_aliases={3: 0, 4: 1},
        scratch_shapes=[
            pltpu.VMEM((NB, BR, D), jnp.bfloat16),
            pltpu.VMEM((1, D), jnp.float32),
            pltpu.VMEM((2, TR, D), jnp.float32),
            pltpu.VMEM((2, TR, D), jnp.float32),
            pltpu.VMEM((2, TR, D), jnp.bfloat16),
            pltpu.VMEM((2, TR, D), jnp.bfloat16),
            pltpu.VMEM((2, TR, D), jnp.float32),
        ]
        + [pltpu.SemaphoreType.DMA((NB,))] * 10
        + [
            pltpu.SemaphoreType.DMA,
            pltpu.SemaphoreType.DMA((2,)),
            pltpu.SemaphoreType.DMA((2,)),
            pltpu.SemaphoreType.DMA((2,)),
            pltpu.SemaphoreType.DMA((2,)),
            pltpu.SemaphoreType.DMA((2,)),
        ],
        compiler_params=pltpu.CompilerParams(collective_id=0),
    )(partial, resid, gamma2d, out_in, gbuf_in)
    return out_f32


def kernel(partial, resid, gamma):
    return _fused(
        partial, resid, gamma.reshape(1, D),
        _alloc(jnp.float32), _alloc(jnp.bfloat16),
    )


# device time: 190406 ns/iter; 1.0961x vs baseline; 1.0902x over previous
import jax
import jax.numpy as jnp
from jax import lax
from jax.experimental import pallas as pl
from jax.experimental.pallas import tpu as pltpu

M = 4096
D = 4096
CH = M // 4
NB = 8
BR = CH // NB
TR = BR
H2 = BR // 2
TILE = 128
EPS = 1e-6

_MESH = pl.DeviceIdType.MESH


def _fused(partial, resid, gamma2d):
    def body(
        p_ref, res_ref, g_ref,
        gbuf_ref,
        recv_y, gamma_v, pstage, rstage, ntile, cvt_in,
        ysend, yrecv, xs, xr, zs, zr, fxs, fxr, fzs, fzr,
        gsem, psem, rsem, nsem,
    ):
        my_x = lax.axis_index("x")
        my_y = lax.axis_index("y")
        my_z = lax.axis_index("z")
        nbr_y = (my_x, 1 - my_y, my_z)
        nbr_x = (1 - my_x, my_y, my_z)
        nbr_z = (my_x, my_y, 1 - my_z)

        c0 = (my_x * 2 + my_z) * CH
        cx0 = ((1 - my_x) * 2 + my_z) * CH
        cz0 = (my_x * 2 + (1 - my_z)) * CH
        cd0 = ((1 - my_x) * 2 + (1 - my_z)) * CH

        barrier_sem = pltpu.get_barrier_semaphore()
        for nbr in (nbr_x, nbr_y, nbr_z):
            pl.semaphore_signal(
                barrier_sem, inc=1, device_id=nbr, device_id_type=_MESH
            )
        pl.semaphore_wait(barrier_sem, 3)

        def rcopy(src, dst, ssem, rsem_, nbr):
            return pltpu.make_async_remote_copy(
                src_ref=src, dst_ref=dst, send_sem=ssem, recv_sem=rsem_,
                device_id=nbr, device_id_type=_MESH,
            )

        gdma = pltpu.make_async_copy(g_ref, gamma_v, gsem)
        gdma.start()
        gdma.wait()

        y_rdmas, x_rdmas, z_rdmas = [], [], []
        rds = [None] * NB
        pds = [None] * NB

        def start_pr(b):
            slot = b % 2
            pds[b] = pltpu.make_async_copy(
                p_ref.at[0, pl.ds(c0 + b * BR, BR)], pstage.at[slot], psem.at[slot]
            )
            rds[b] = pltpu.make_async_copy(
                res_ref.at[pl.ds(c0 + b * BR, BR)], rstage.at[slot], rsem.at[slot]
            )
            pds[b].start()
            rds[b].start()

        def stage_cast_send(b):
            slot = b % 2
            if b >= 2:
                y_rdmas[b - 2].wait_send()
            pds[b].wait()
            cvt_in[slot] = pstage[slot].astype(jnp.bfloat16)
            r = rcopy(
                cvt_in.at[slot], recv_y.at[b],
                ysend.at[b], yrecv.at[b], nbr_y,
            )
            r.start()
            y_rdmas.append(r)

        def finish_block(b):
            slot = b % 2
            y_rdmas[b].wait_recv()
            rds[b].wait()
            y32 = pstage[slot] + recv_y[b].astype(jnp.float32) + rstage[slot]
            if b + 2 < NB:
                start_pr(b + 2)
            ms = jnp.mean(y32 * y32, axis=-1, keepdims=True)
            norm = y32 * lax.rsqrt(ms + EPS) * gamma_v[...]
            ntile[slot] = norm.astype(jnp.bfloat16)
            nd = pltpu.make_async_copy(
                ntile.at[slot], gbuf_ref.at[pl.ds(c0 + b * BR, BR)], nsem.at[slot]
            )
            nd.start()
            nd.wait()
            rx = rcopy(
                gbuf_ref.at[pl.ds(c0 + b * BR, BR)],
                gbuf_ref.at[pl.ds(c0 + b * BR, BR)],
                xs.at[b], xr.at[b], nbr_x,
            )
            rz = rcopy(
                gbuf_ref.at[pl.ds(c0 + b * BR, BR)],
                gbuf_ref.at[pl.ds(c0 + b * BR, BR)],
                zs.at[b], zr.at[b], nbr_z,
            )
            rx.start()
            rz.start()
            x_rdmas.append(rx)
            z_rdmas.append(rz)

        start_pr(0)
        start_pr(1)
        stage_cast_send(0)
        for b in range(1, NB):
            stage_cast_send(b)
            finish_block(b - 1)
        finish_block(NB - 1)

        fx_rdmas, fz_rdmas = [], []
        for b in range(NB):
            rcopy(
                gbuf_ref.at[pl.ds(cz0 + b * BR, BR)],
                gbuf_ref.at[pl.ds(cz0 + b * BR, BR)],
                zs.at[b], zr.at[b], nbr_z,
            ).wait_recv()
            fx = rcopy(
                gbuf_ref.at[pl.ds(cz0 + b * BR, H2)],
                gbuf_ref.at[pl.ds(cz0 + b * BR, H2)],
                fxs.at[b], fxr.at[b], nbr_x,
            )
            fx.start()
            fx_rdmas.append(fx)
            rcopy(
                gbuf_ref.at[pl.ds(cx0 + b * BR, BR)],
                gbuf_ref.at[pl.ds(cx0 + b * BR, BR)],
                xs.at[b], xr.at[b], nbr_x,
            ).wait_recv()
            fz = rcopy(
                gbuf_ref.at[pl.ds(cx0 + b * BR + H2, H2)],
                gbuf_ref.at[pl.ds(cx0 + b * BR + H2, H2)],
                fzs.at[b], fzr.at[b], nbr_z,
            )
            fz.start()
            fz_rdmas.append(fz)

        for b in range(NB):
            rcopy(
                gbuf_ref.at[pl.ds(cd0 + b * BR, H2)],
                gbuf_ref.at[pl.ds(cd0 + b * BR, H2)],
                fxs.at[b], fxr.at[b], nbr_x,
            ).wait_recv()
            rcopy(
                gbuf_ref.at[pl.ds(cd0 + b * BR + H2, H2)],
                gbuf_ref.at[pl.ds(cd0 + b * BR + H2, H2)],
                fzs.at[b], fzr.at[b], nbr_z,
            ).wait_recv()

        y_rdmas[NB - 2].wait_send()
        y_rdmas[NB - 1].wait_send()
        for r in x_rdmas + z_rdmas + fx_rdmas + fz_rdmas:
            r.wait_send()

    return pl.pallas_call(
        body,
        out_shape=jax.ShapeDtypeStruct((M, D), jnp.bfloat16),
        in_specs=[pl.BlockSpec(memory_space=pl.ANY)] * 3,
        out_specs=pl.BlockSpec(memory_space=pl.ANY),
        scratch_shapes=[
            pltpu.VMEM((NB, BR, D), jnp.bfloat16),
            pltpu.VMEM((1, D), jnp.float32),
            pltpu.VMEM((2, TR, D), jnp.float32),
            pltpu.VMEM((2, TR, D), jnp.float32),
            pltpu.VMEM((2, TR, D), jnp.bfloat16),
            pltpu.VMEM((2, TR, D), jnp.bfloat16),
        ]
        + [pltpu.SemaphoreType.DMA((NB,))] * 10
        + [
            pltpu.SemaphoreType.DMA,
            pltpu.SemaphoreType.DMA((2,)),
            pltpu.SemaphoreType.DMA((2,)),
            pltpu.SemaphoreType.DMA((2,)),
        ],
        compiler_params=pltpu.CompilerParams(collective_id=0),
    )(partial, resid, gamma2d)


def kernel(partial, resid, gamma):
    return _fused(partial, resid, gamma.reshape(1, D)).astype(jnp.float32)
